# baseline (speedup 1.0000x reference)
import functools

import jax
import jax.numpy as jnp
from jax import lax
from jax.experimental import pallas as pl
from jax.experimental.pallas import tpu as pltpu
from jax.experimental.pallas import tpu_sc as plsc

N_NODES = 1_000_000
DIM = 32
B0, B1 = 4096, 200

NC, NS = 2, 16
NW = NC * NS
IBLK = B0 // NW

JH = 40
NH = B1 // JH
G = 5
NG = B1 // G
LANES = 16


def _sc_gather(idx, table):
    mesh = plsc.VectorSubcoreMesh(
        core_axis_name="c", subcore_axis_name="s",
        num_cores=NC, num_subcores=NS)

    @functools.partial(
        pl.kernel,
        out_type=jax.ShapeDtypeStruct((B1, DIM // 8, NW, 8, IBLK),
                                      jnp.float32),
        mesh=mesh,
        scratch_types=[
            pltpu.VMEM((IBLK, JH), jnp.int32),
            pltpu.VMEM((B1, IBLK), jnp.int32),
            pltpu.VMEM((G * IBLK, DIM), jnp.float32),
            pltpu.VMEM((G * IBLK, DIM), jnp.float32),
            pltpu.VMEM((G, DIM // 8, 8, IBLK + 1), jnp.float32),
            pltpu.VMEM((G, DIM // 8, 8, IBLK + 1), jnp.float32),
            pltpu.SemaphoreType.DMA,
            pltpu.SemaphoreType.DMA,
            pltpu.SemaphoreType.DMA,
            pltpu.SemaphoreType.DMA,
        ],
        compiler_params=pltpu.CompilerParams(
            use_tc_tiling_on_sc=False, needs_layout_passes=False),
    )
    def k(idx_hbm, table_hbm, out_hbm, idx_raw, idx_t, rows0, rows1,
          t0, t1, gsem0, gsem1, ssem0, ssem1):
        wid = lax.axis_index("s") * NC + lax.axis_index("c")
        i0 = wid * IBLK

        lane = lax.iota(jnp.int32, LANES)
        rowvec = [lane + LANES * c for c in range(IBLK // LANES)]

        def stage_idx(h):
            pltpu.sync_copy(
                idx_hbm.at[pl.ds(i0, IBLK), pl.ds(h * JH, JH)], idx_raw)

            @plsc.parallel_loop(0, JH, unroll=4)
            def _(jj, h=h):
                col = jnp.full((LANES,), jj, jnp.int32)
                for c in range(IBLK // LANES):
                    v = plsc.load_gather(idx_raw, [rowvec[c], col])
                    idx_t[h * JH + jj, pl.ds(LANES * c, LANES)] = v

        def fire(g, rows, gsem):
            for jg in range(G):
                pltpu.async_copy(
                    table_hbm.at[idx_t.at[g * G + jg]],
                    rows.at[pl.ds(jg * IBLK, IBLK)],
                    gsem)

        def drain(rows, gsem):
            pltpu.make_async_copy(
                table_hbm.at[pl.ds(0, G * IBLK)], rows, gsem).wait()

        dbv = [lane // 8 + 2 * h for h in range(DIM // LANES)]
        rv8 = lane % 8
        jsplat = [jnp.full((LANES,), g, jnp.int32) for g in range(G)]

        def transpose(rows, tbuf):
            for jg in range(G):
                base = jg * IBLK

                @plsc.parallel_loop(0, IBLK, unroll=16)
                def _(l, jg=jg, base=base):
                    ls = jnp.full((LANES,), l, jnp.int32)
                    for h in range(DIM // LANES):
                        v = rows[base + l, pl.ds(LANES * h, LANES)]
                        plsc.store_scatter(
                            tbuf, [jsplat[jg], dbv[h], rv8, ls], v)

        def start_store(g, tbuf, ssem):
            pltpu.async_copy(
                tbuf.at[:, :, :, pl.ds(0, IBLK)],
                out_hbm.at[pl.ds(g * G, G), :, wid], ssem)

        def wait_store(tbuf, ssem):
            pltpu.make_async_copy(
                tbuf.at[:, :, :, pl.ds(0, IBLK)],
                out_hbm.at[pl.ds(0, G), :, 0], ssem).wait()

        stage_idx(0)
        fire(0, rows0, gsem0)
        fire(1, rows1, gsem1)
        for h in range(1, NH):
            stage_idx(h)
        drain(rows0, gsem0)
        transpose(rows0, t0)
        start_store(0, t0, ssem0)
        fire(2, rows0, gsem0)
        drain(rows1, gsem1)
        transpose(rows1, t1)
        start_store(1, t1, ssem1)

        def body(ss, carry):
            g0 = 2 * ss
            g1 = g0 + 1
            fire(g1, rows1, gsem1)
            drain(rows0, gsem0)
            wait_store(t0, ssem0)
            transpose(rows0, t0)
            start_store(g0, t0, ssem0)
            fire(g1 + 1, rows0, gsem0)
            drain(rows1, gsem1)
            wait_store(t1, ssem1)
            transpose(rows1, t1)
            start_store(g1, t1, ssem1)
            return carry

        lax.fori_loop(1, NG // 2 - 1, body, 0)

        fire(NG - 1, rows1, gsem1)
        drain(rows0, gsem0)
        wait_store(t0, ssem0)
        transpose(rows0, t0)
        start_store(NG - 2, t0, ssem0)
        drain(rows1, gsem1)
        wait_store(t1, ssem1)
        transpose(rows1, t1)
        start_store(NG - 1, t1, ssem1)
        wait_store(t0, ssem0)
        wait_store(t1, ssem1)

    return k(idx, table)


def kernel(indices, embeddings):
    out5 = _sc_gather(indices, embeddings)
    return out5.transpose(2, 4, 0, 1, 3).reshape(B0, B1, DIM)


# speedup vs baseline: 1.0039x; 1.0039x over previous
import functools

import jax
import jax.numpy as jnp
from jax import lax
from jax.experimental import pallas as pl
from jax.experimental.pallas import tpu as pltpu
from jax.experimental.pallas import tpu_sc as plsc

N_NODES = 1_000_000
DIM = 32
B0, B1 = 4096, 200

NC, NS = 2, 16
NW = NC * NS
IBLK = B0 // NW

JH = 40
NH = B1 // JH
G = 5
NG = B1 // G
LANES = 16


def _sc_gather(idx, table):
    mesh = plsc.VectorSubcoreMesh(
        core_axis_name="c", subcore_axis_name="s",
        num_cores=NC, num_subcores=NS)

    @functools.partial(
        pl.kernel,
        out_type=jax.ShapeDtypeStruct((B1, DIM // 8, NW, 8, IBLK),
                                      jnp.float32),
        mesh=mesh,
        scratch_types=[
            pltpu.VMEM((IBLK, JH), jnp.int32),
            pltpu.VMEM((B1, IBLK), jnp.int32),
            pltpu.VMEM((G * IBLK, DIM), jnp.float32),
            pltpu.VMEM((G * IBLK, DIM), jnp.float32),
            pltpu.VMEM((G, DIM // 8, 8, IBLK + 1), jnp.float32),
            pltpu.VMEM((G, DIM // 8, 8, IBLK + 1), jnp.float32),
            pltpu.SemaphoreType.DMA,
            pltpu.SemaphoreType.DMA,
            pltpu.SemaphoreType.DMA,
            pltpu.SemaphoreType.DMA,
        ],
        compiler_params=pltpu.CompilerParams(
            use_tc_tiling_on_sc=False, needs_layout_passes=False),
    )
    def k(idx_hbm, table_hbm, out_hbm, idx_raw, idx_t, rows0, rows1,
          t0, t1, gsem0, gsem1, ssem0, ssem1):
        wid = lax.axis_index("s") * NC + lax.axis_index("c")
        i0 = wid * IBLK

        lane = lax.iota(jnp.int32, LANES)
        rowvec = [lane + LANES * c for c in range(IBLK // LANES)]

        def stage_idx(h):
            pltpu.sync_copy(
                idx_hbm.at[pl.ds(i0, IBLK), pl.ds(h * JH, JH)], idx_raw)

            @plsc.parallel_loop(0, JH, unroll=4)
            def _(jj, h=h):
                col = jnp.full((LANES,), jj, jnp.int32)
                for c in range(IBLK // LANES):
                    v = plsc.load_gather(idx_raw, [rowvec[c], col])
                    idx_t[h * JH + jj, pl.ds(LANES * c, LANES)] = v

        def fire(g, rows, gsem):
            for jg in range(G):
                pltpu.async_copy(
                    table_hbm.at[idx_t.at[g * G + jg]],
                    rows.at[pl.ds(jg * IBLK, IBLK)],
                    gsem)

        def drain(rows, gsem):
            pltpu.make_async_copy(
                table_hbm.at[pl.ds(0, G * IBLK)], rows, gsem).wait()

        dbv = [lane // 8 + 2 * h for h in range(DIM // LANES)]
        rv8 = lane % 8
        jsplat = [jnp.full((LANES,), g, jnp.int32) for g in range(G)]

        def transpose(rows, tbuf):
            for jg in range(G):
                base = jg * IBLK

                @plsc.parallel_loop(0, IBLK, unroll=8)
                def _(l, jg=jg, base=base):
                    ls = jnp.full((LANES,), l, jnp.int32)
                    for h in range(DIM // LANES):
                        v = rows[base + l, pl.ds(LANES * h, LANES)]
                        plsc.store_scatter(
                            tbuf, [jsplat[jg], dbv[h], rv8, ls], v)

        def start_store(g, tbuf, ssem):
            pltpu.async_copy(
                tbuf.at[:, :, :, pl.ds(0, IBLK)],
                out_hbm.at[pl.ds(g * G, G), :, wid], ssem)

        def wait_store(tbuf, ssem):
            pltpu.make_async_copy(
                tbuf.at[:, :, :, pl.ds(0, IBLK)],
                out_hbm.at[pl.ds(0, G), :, 0], ssem).wait()

        stage_idx(0)
        fire(0, rows0, gsem0)
        fire(1, rows1, gsem1)
        for h in range(1, NH):
            stage_idx(h)
        drain(rows0, gsem0)
        transpose(rows0, t0)
        start_store(0, t0, ssem0)
        fire(2, rows0, gsem0)
        drain(rows1, gsem1)
        transpose(rows1, t1)
        start_store(1, t1, ssem1)

        def body(ss, carry):
            g0 = 2 * ss
            g1 = g0 + 1
            fire(g1, rows1, gsem1)
            drain(rows0, gsem0)
            wait_store(t0, ssem0)
            transpose(rows0, t0)
            start_store(g0, t0, ssem0)
            fire(g1 + 1, rows0, gsem0)
            drain(rows1, gsem1)
            wait_store(t1, ssem1)
            transpose(rows1, t1)
            start_store(g1, t1, ssem1)
            return carry

        lax.fori_loop(1, NG // 2 - 1, body, 0)

        fire(NG - 1, rows1, gsem1)
        drain(rows0, gsem0)
        wait_store(t0, ssem0)
        transpose(rows0, t0)
        start_store(NG - 2, t0, ssem0)
        drain(rows1, gsem1)
        wait_store(t1, ssem1)
        transpose(rows1, t1)
        start_store(NG - 1, t1, ssem1)
        wait_store(t0, ssem0)
        wait_store(t1, ssem1)

    return k(idx, table)


def kernel(indices, embeddings):
    out5 = _sc_gather(indices, embeddings)
    return out5.transpose(2, 4, 0, 1, 3).reshape(B0, B1, DIM)
